# baseline (speedup 1.0000x reference)
import functools

import jax
import jax.numpy as jnp
from jax import lax
from jax.experimental import pallas as pl
from jax.experimental.pallas import tpu as pltpu
from jax.experimental.pallas import tpu_sc as plsc

V = 1000000
D = 64
L_CTX = 200
CBLK = 32768
NBLK = (V + CBLK - 1) // CBLK

SC_BLOCKS = 12
TC_BLOCKS = NBLK - SC_BLOCKS
SC_COLS = SC_BLOCKS * CBLK
NW = 32
TCOLS = SC_COLS // NW
CH = 512
NCH = TCOLS // CH


def _gather_body(ctx_ref, embt_ref, e_ref):
    i = pl.program_id(0)

    @pl.when(i == 0)
    def _():
        e_ref[...] = jnp.zeros_like(e_ref)

    lane = ctx_ref[i] % 128
    mask = lax.broadcasted_iota(jnp.int32, (D, 128), 1) == lane
    e_ref[...] += jnp.where(mask, embt_ref[...], 0.0)

    @pl.when(i == L_CTX - 1)
    def _():
        tot = jnp.sum(e_ref[...], axis=1, keepdims=True) * (1.0 / L_CTX)
        e_ref[...] = jnp.broadcast_to(tot, (D, 128))


def _matvec_body(e_ref, wt_ref, b_ref, z_ref, stats_ref, m_ref, s_ref):
    g = pl.program_id(0)

    @pl.when(g == 0)
    def _():
        m_ref[0] = -jnp.inf
        s_ref[0] = 0.0

    z = lax.dot_general(
        e_ref[...], wt_ref[...], (((0,), (0,)), ((), ())),
        preferred_element_type=jnp.float32,
    )
    z = z + b_ref[...][None, :]
    col = (SC_BLOCKS + g) * CBLK + lax.broadcasted_iota(
        jnp.int32, (1, CBLK), 1)
    z = jnp.where(col < V, z, -jnp.inf)
    z_ref[...] = z[0]
    m_old = m_ref[0]
    m_new = jnp.maximum(m_old, jnp.max(z))
    bsum = jnp.sum(jnp.exp(z - m_new))
    s_ref[0] = s_ref[0] * jnp.exp(m_old - m_new) + bsum
    m_ref[0] = m_new

    @pl.when(g == TC_BLOCKS - 1)
    def _():
        stats_ref[0] = m_ref[0]
        stats_ref[1] = s_ref[0]


def _norm_body(z_tc_ref, z_sc_ref, statm_ref, stats_ref, stats_tc_ref,
               o_ref):
    g = pl.program_id(0)
    msc = statm_ref[...]
    ssc = stats_ref[...]
    m_tc = stats_tc_ref[0]
    s_tc = stats_tc_ref[1]
    m_all = jnp.maximum(jnp.max(msc), m_tc)
    stot = (jnp.sum(ssc * jnp.exp(msc - m_all))
            + s_tc * jnp.exp(m_tc - m_all))
    lse = m_all + jnp.log(stot)
    z = jnp.where(g < SC_BLOCKS, z_sc_ref[...], z_tc_ref[...])
    o_ref[...] = z - lse


def _sc_matvec(wt, b, ebc_in):
    mesh = plsc.VectorSubcoreMesh(core_axis_name="c", subcore_axis_name="s")

    @functools.partial(
        pl.kernel,
        mesh=mesh,
        out_type=[
            jax.ShapeDtypeStruct((SC_COLS,), jnp.float32),
            jax.ShapeDtypeStruct((NW, 16), jnp.float32),
            jax.ShapeDtypeStruct((NW, 16), jnp.float32),
        ],
        scratch_types=[
            pltpu.VMEM((D, CH), jnp.float32),
            pltpu.VMEM((D, CH), jnp.float32),
            pltpu.VMEM((TCOLS,), jnp.float32),
            pltpu.VMEM((TCOLS,), jnp.float32),
            pltpu.VMEM((D, 16), jnp.float32),
            pltpu.VMEM((16,), jnp.float32),
            pltpu.VMEM((16,), jnp.float32),
            pltpu.VMEM((1, 16), jnp.float32),
            pltpu.SemaphoreType.DMA,
            pltpu.SemaphoreType.DMA,
        ],
    )
    def k(wt_hbm, b_hbm, ebc_hbm, z_hbm, statm_hbm, stats_hbm,
          buf0, buf1, zv, bv, ebc, mv, sv, statv, sem0, sem1):
        c = lax.axis_index("c")
        s = lax.axis_index("s")
        wid = s * 2 + c
        base = wid * TCOLS

        pltpu.sync_copy(ebc_hbm, ebc)
        pltpu.sync_copy(b_hbm.at[pl.ds(base, TCOLS)], bv)

        mv[...] = jnp.full((16,), -jnp.inf, jnp.float32)

        pltpu.async_copy(wt_hbm.at[:, pl.ds(base, CH)], buf0, sem0)
        pltpu.async_copy(wt_hbm.at[:, pl.ds(base + CH, CH)], buf1, sem1)

        def compute_chunk(ci, buf):
            cb = ci * CH

            def c16_body(t, carry):
                off = cb + t * 16
                a0 = bv[pl.ds(off, 16)]
                a1 = jnp.zeros((16,), jnp.float32)
                a2 = jnp.zeros((16,), jnp.float32)
                a3 = jnp.zeros((16,), jnp.float32)
                for d in range(0, D, 4):
                    sl = pl.ds(t * 16, 16)
                    a0 = a0 + buf[d, sl] * ebc[d, :]
                    a1 = a1 + buf[d + 1, sl] * ebc[d + 1, :]
                    a2 = a2 + buf[d + 2, sl] * ebc[d + 2, :]
                    a3 = a3 + buf[d + 3, sl] * ebc[d + 3, :]
                acc = (a0 + a1) + (a2 + a3)
                zv[pl.ds(off, 16)] = acc
                mv[...] = jnp.maximum(mv[...], acc)
                return carry

            lax.fori_loop(0, CH // 16, c16_body, 0)

        def pair_body(j, carry):
            c0 = 2 * j
            c1 = 2 * j + 1
            pltpu.make_async_copy(
                wt_hbm.at[:, pl.ds(0, CH)], buf0, sem0).wait()
            compute_chunk(c0, buf0)

            @pl.when(c0 + 2 < NCH)
            def _():
                pltpu.async_copy(
                    wt_hbm.at[:, pl.ds(base + (c0 + 2) * CH, CH)],
                    buf0, sem0)

            pltpu.make_async_copy(
                wt_hbm.at[:, pl.ds(0, CH)], buf1, sem1).wait()
            compute_chunk(c1, buf1)

            @pl.when(c1 + 2 < NCH)
            def _():
                pltpu.async_copy(
                    wt_hbm.at[:, pl.ds(base + (c1 + 2) * CH, CH)],
                    buf1, sem1)

            return carry

        lax.fori_loop(0, NCH // 2, pair_body, 0)

        m_b = mv[...]
        sv[...] = jnp.zeros((16,), jnp.float32)

        def s_body(t, carry):
            sv[...] += jnp.exp(zv[pl.ds(t * 16, 16)] - m_b)
            return carry

        lax.fori_loop(0, TCOLS // 16, s_body, 0)

        pltpu.sync_copy(zv, z_hbm.at[pl.ds(base, TCOLS)])
        statv[0, :] = mv[...]
        pltpu.sync_copy(statv, statm_hbm.at[pl.ds(wid, 1)])
        statv[0, :] = sv[...]
        pltpu.sync_copy(statv, stats_hbm.at[pl.ds(wid, 1)])

    return k(wt, b, ebc_in)


def kernel(context, emb, W, b):
    ctx = context.astype(jnp.int32)
    embt = emb.T
    wt = W.T

    e_wide = pl.pallas_call(
        _gather_body,
        grid_spec=pltpu.PrefetchScalarGridSpec(
            num_scalar_prefetch=1,
            grid=(L_CTX,),
            in_specs=[pl.BlockSpec((D, 128), lambda i, c: (0, c[i] // 128))],
            out_specs=pl.BlockSpec((D, 128), lambda i, c: (0, 0)),
        ),
        out_shape=jax.ShapeDtypeStruct((D, 128), jnp.float32),
    )(ctx, embt)
    e2 = e_wide[:, :1]

    ebc_in = jnp.broadcast_to(e_wide[:, :1], (D, 16))
    z_sc, statm_sc, stats_sc = _sc_matvec(wt, b, ebc_in)

    z_tc, stats_tc = pl.pallas_call(
        _matvec_body,
        grid=(TC_BLOCKS,),
        in_specs=[
            pl.BlockSpec((D, 1), lambda g: (0, 0)),
            pl.BlockSpec((D, CBLK), lambda g: (0, SC_BLOCKS + g)),
            pl.BlockSpec((CBLK,), lambda g: (SC_BLOCKS + g,)),
        ],
        out_specs=[
            pl.BlockSpec((CBLK,), lambda g: (g,)),
            pl.BlockSpec(block_shape=(2,), index_map=lambda g: (0,),
                         memory_space=pltpu.SMEM),
        ],
        out_shape=[
            jax.ShapeDtypeStruct((TC_BLOCKS * CBLK,), jnp.float32),
            jax.ShapeDtypeStruct((2,), jnp.float32),
        ],
        scratch_shapes=[
            pltpu.SMEM((1,), jnp.float32),
            pltpu.SMEM((1,), jnp.float32),
        ],
    )(e2, wt, b)

    out = pl.pallas_call(
        _norm_body,
        grid=(NBLK,),
        in_specs=[
            pl.BlockSpec((CBLK,),
                         lambda g: (jnp.clip(g - SC_BLOCKS, 0,
                                             TC_BLOCKS - 1),)),
            pl.BlockSpec((CBLK,),
                         lambda g: (jnp.minimum(g, SC_BLOCKS - 1),)),
            pl.BlockSpec((NW, 16), lambda g: (0, 0)),
            pl.BlockSpec((NW, 16), lambda g: (0, 0)),
            pl.BlockSpec(block_shape=(2,), index_map=lambda g: (0,),
                         memory_space=pltpu.SMEM),
        ],
        out_specs=pl.BlockSpec((CBLK,), lambda g: (g,)),
        out_shape=jax.ShapeDtypeStruct((V,), jnp.float32),
    )(z_tc, z_sc, statm_sc, stats_sc, stats_tc)
    return out


# speedup vs baseline: 1.0879x; 1.0879x over previous
import functools

import jax
import jax.numpy as jnp
from jax import lax
from jax.experimental import pallas as pl
from jax.experimental.pallas import tpu as pltpu
from jax.experimental.pallas import tpu_sc as plsc

V = 1000000
D = 64
L_CTX = 200
CBLK = 32768
NBLK = (V + CBLK - 1) // CBLK

SC_BLOCKS = 12
TC_BLOCKS = NBLK - SC_BLOCKS
SC_COLS = SC_BLOCKS * CBLK
NW = 32
TCOLS = SC_COLS // NW
CH = 512
NCH = TCOLS // CH


def _gather_body(ctx_ref, embt_ref, e_ref):
    i = pl.program_id(0)

    @pl.when(i == 0)
    def _():
        e_ref[...] = jnp.zeros_like(e_ref)

    lane = ctx_ref[i] % 128
    mask = lax.broadcasted_iota(jnp.int32, (D, 128), 1) == lane
    e_ref[...] += jnp.where(mask, embt_ref[...], 0.0)

    @pl.when(i == L_CTX - 1)
    def _():
        tot = jnp.sum(e_ref[...], axis=1, keepdims=True) * (1.0 / L_CTX)
        e_ref[...] = jnp.broadcast_to(tot, (D, 128))


def _matvec_body(e_ref, wt_ref, b_ref, z_ref, stats_ref, m_ref, s_ref):
    g = pl.program_id(0)

    @pl.when(g == 0)
    def _():
        m_ref[0] = -jnp.inf
        s_ref[0] = 0.0

    z = lax.dot_general(
        e_ref[...], wt_ref[...], (((0,), (0,)), ((), ())),
        preferred_element_type=jnp.float32,
    )
    z = z + b_ref[...][None, :]
    col = (SC_BLOCKS + g) * CBLK + lax.broadcasted_iota(
        jnp.int32, (1, CBLK), 1)
    z = jnp.where(col < V, z, -jnp.inf)
    z_ref[...] = z[0]
    m_old = m_ref[0]
    m_new = jnp.maximum(m_old, jnp.max(z))
    bsum = jnp.sum(jnp.exp(z - m_new))
    s_ref[0] = s_ref[0] * jnp.exp(m_old - m_new) + bsum
    m_ref[0] = m_new

    @pl.when(g == TC_BLOCKS - 1)
    def _():
        stats_ref[0] = m_ref[0]
        stats_ref[1] = s_ref[0]


def _norm_body(z_tc_ref, z_sc_ref, statm_ref, stats_ref, stats_tc_ref,
               o_ref):
    g = pl.program_id(0)
    msc = statm_ref[...]
    ssc = stats_ref[...]
    m_tc = stats_tc_ref[0]
    s_tc = stats_tc_ref[1]
    m_all = jnp.maximum(jnp.max(msc), m_tc)
    stot = (jnp.sum(ssc * jnp.exp(msc - m_all))
            + s_tc * jnp.exp(m_tc - m_all))
    lse = m_all + jnp.log(stot)
    z = jnp.where(g < SC_BLOCKS, z_sc_ref[...], z_tc_ref[...])
    o_ref[...] = z - lse


def _sc_matvec(wt, b, ebc_in):
    mesh = plsc.VectorSubcoreMesh(core_axis_name="c", subcore_axis_name="s")

    @functools.partial(
        pl.kernel,
        mesh=mesh,
        out_type=[
            jax.ShapeDtypeStruct((SC_COLS,), jnp.float32),
            jax.ShapeDtypeStruct((NW, 16), jnp.float32),
            jax.ShapeDtypeStruct((NW, 16), jnp.float32),
        ],
        scratch_types=[
            pltpu.VMEM((D, CH), jnp.float32),
            pltpu.VMEM((D, CH), jnp.float32),
            pltpu.VMEM((TCOLS,), jnp.float32),
            pltpu.VMEM((TCOLS,), jnp.float32),
            pltpu.VMEM((D, 16), jnp.float32),
            pltpu.VMEM((16,), jnp.float32),
            pltpu.VMEM((16,), jnp.float32),
            pltpu.VMEM((1, 16), jnp.float32),
            pltpu.SemaphoreType.DMA,
            pltpu.SemaphoreType.DMA,
        ],
    )
    def k(wt_hbm, b_hbm, ebc_hbm, z_hbm, statm_hbm, stats_hbm,
          buf0, buf1, zv, bv, ebc, mv, sv, statv, sem0, sem1):
        c = lax.axis_index("c")
        s = lax.axis_index("s")
        wid = s * 2 + c
        base = wid * TCOLS

        pltpu.sync_copy(ebc_hbm, ebc)
        pltpu.sync_copy(b_hbm.at[pl.ds(base, TCOLS)], bv)

        mv[...] = jnp.full((16,), -jnp.inf, jnp.float32)

        pltpu.async_copy(wt_hbm.at[:, pl.ds(base, CH)], buf0, sem0)
        pltpu.async_copy(wt_hbm.at[:, pl.ds(base + CH, CH)], buf1, sem1)

        def compute_chunk(ci, buf):
            cb = ci * CH
            for half in range(2):
                d0 = half * (D // 2)
                es = tuple(ebc[d0 + d, :] for d in range(D // 2))

                def c16_body(t, carry, d0=d0, es=es):
                    off = cb + t * 16
                    sl = pl.ds(t * 16, 16)
                    if d0 == 0:
                        a0 = bv[pl.ds(off, 16)]
                    else:
                        a0 = zv[pl.ds(off, 16)]
                    a1 = jnp.zeros((16,), jnp.float32)
                    a2 = jnp.zeros((16,), jnp.float32)
                    a3 = jnp.zeros((16,), jnp.float32)
                    for d in range(0, D // 2, 4):
                        a0 = a0 + buf[d0 + d, sl] * es[d]
                        a1 = a1 + buf[d0 + d + 1, sl] * es[d + 1]
                        a2 = a2 + buf[d0 + d + 2, sl] * es[d + 2]
                        a3 = a3 + buf[d0 + d + 3, sl] * es[d + 3]
                    acc = (a0 + a1) + (a2 + a3)
                    zv[pl.ds(off, 16)] = acc
                    if d0 != 0:
                        mv[...] = jnp.maximum(mv[...], acc)
                    return carry

                lax.fori_loop(0, CH // 16, c16_body, 0)

        def pair_body(j, carry):
            c0 = 2 * j
            c1 = 2 * j + 1
            pltpu.make_async_copy(
                wt_hbm.at[:, pl.ds(0, CH)], buf0, sem0).wait()
            compute_chunk(c0, buf0)

            @pl.when(c0 + 2 < NCH)
            def _():
                pltpu.async_copy(
                    wt_hbm.at[:, pl.ds(base + (c0 + 2) * CH, CH)],
                    buf0, sem0)

            pltpu.make_async_copy(
                wt_hbm.at[:, pl.ds(0, CH)], buf1, sem1).wait()
            compute_chunk(c1, buf1)

            @pl.when(c1 + 2 < NCH)
            def _():
                pltpu.async_copy(
                    wt_hbm.at[:, pl.ds(base + (c1 + 2) * CH, CH)],
                    buf1, sem1)

            return carry

        lax.fori_loop(0, NCH // 2, pair_body, 0)

        m_b = mv[...]
        sv[...] = jnp.zeros((16,), jnp.float32)

        def s_body(t, carry):
            sv[...] += jnp.exp(zv[pl.ds(t * 16, 16)] - m_b)
            return carry

        lax.fori_loop(0, TCOLS // 16, s_body, 0)

        pltpu.sync_copy(zv, z_hbm.at[pl.ds(base, TCOLS)])
        statv[0, :] = mv[...]
        pltpu.sync_copy(statv, statm_hbm.at[pl.ds(wid, 1)])
        statv[0, :] = sv[...]
        pltpu.sync_copy(statv, stats_hbm.at[pl.ds(wid, 1)])

    return k(wt, b, ebc_in)


def kernel(context, emb, W, b):
    ctx = context.astype(jnp.int32)
    embt = emb.T
    wt = W.T

    e_wide = pl.pallas_call(
        _gather_body,
        grid_spec=pltpu.PrefetchScalarGridSpec(
            num_scalar_prefetch=1,
            grid=(L_CTX,),
            in_specs=[pl.BlockSpec((D, 128), lambda i, c: (0, c[i] // 128))],
            out_specs=pl.BlockSpec((D, 128), lambda i, c: (0, 0)),
        ),
        out_shape=jax.ShapeDtypeStruct((D, 128), jnp.float32),
    )(ctx, embt)
    e2 = e_wide[:, :1]

    ebc_in = jnp.broadcast_to(e_wide[:, :1], (D, 16))
    z_sc, statm_sc, stats_sc = _sc_matvec(wt, b, ebc_in)

    z_tc, stats_tc = pl.pallas_call(
        _matvec_body,
        grid=(TC_BLOCKS,),
        in_specs=[
            pl.BlockSpec((D, 1), lambda g: (0, 0)),
            pl.BlockSpec((D, CBLK), lambda g: (0, SC_BLOCKS + g)),
            pl.BlockSpec((CBLK,), lambda g: (SC_BLOCKS + g,)),
        ],
        out_specs=[
            pl.BlockSpec((CBLK,), lambda g: (g,)),
            pl.BlockSpec(block_shape=(2,), index_map=lambda g: (0,),
                         memory_space=pltpu.SMEM),
        ],
        out_shape=[
            jax.ShapeDtypeStruct((TC_BLOCKS * CBLK,), jnp.float32),
            jax.ShapeDtypeStruct((2,), jnp.float32),
        ],
        scratch_shapes=[
            pltpu.SMEM((1,), jnp.float32),
            pltpu.SMEM((1,), jnp.float32),
        ],
    )(e2, wt, b)

    out = pl.pallas_call(
        _norm_body,
        grid=(NBLK,),
        in_specs=[
            pl.BlockSpec((CBLK,),
                         lambda g: (jnp.clip(g - SC_BLOCKS, 0,
                                             TC_BLOCKS - 1),)),
            pl.BlockSpec((CBLK,),
                         lambda g: (jnp.minimum(g, SC_BLOCKS - 1),)),
            pl.BlockSpec((NW, 16), lambda g: (0, 0)),
            pl.BlockSpec((NW, 16), lambda g: (0, 0)),
            pl.BlockSpec(block_shape=(2,), index_map=lambda g: (0,),
                         memory_space=pltpu.SMEM),
        ],
        out_specs=pl.BlockSpec((CBLK,), lambda g: (g,)),
        out_shape=jax.ShapeDtypeStruct((V,), jnp.float32),
    )(z_tc, z_sc, statm_sc, stats_sc, stats_tc)
    return out


# speedup vs baseline: 2.0779x; 1.9099x over previous
import jax
import jax.numpy as jnp
from jax import lax
from jax.experimental import pallas as pl
from jax.experimental.pallas import tpu as pltpu

V = 1000000
D = 64
L_CTX = 200
KG = 8
NG = L_CTX // KG
CBLK = 65536
NBLK = (V + CBLK - 1) // CBLK


def _gather_body(ctx_ref, *refs):
    embt_refs = refs[:KG]
    e_ref = refs[KG]
    i = pl.program_id(0)

    @pl.when(i == 0)
    def _():
        e_ref[...] = jnp.zeros_like(e_ref)

    col = lax.broadcasted_iota(jnp.int32, (D, 128), 1)
    acc = e_ref[...]
    for j in range(KG):
        lane = ctx_ref[KG * i + j] % 128
        acc = acc + jnp.where(col == lane, embt_refs[j][...], 0.0)
    e_ref[...] = acc

    @pl.when(i == NG - 1)
    def _():
        tot = jnp.sum(e_ref[...], axis=1, keepdims=True) * (1.0 / L_CTX)
        e_ref[...] = jnp.broadcast_to(tot, (D, 128))


def _matvec_body(e_ref, wt_ref, b_ref, z_ref, lse_ref, m_ref, s_ref):
    g = pl.program_id(0)

    @pl.when(g == 0)
    def _():
        m_ref[0] = -jnp.inf
        s_ref[0] = 0.0

    z = lax.dot_general(
        e_ref[...], wt_ref[...], (((0,), (0,)), ((), ())),
        preferred_element_type=jnp.float32,
    )
    z = z + b_ref[...][None, :]
    col = g * CBLK + lax.broadcasted_iota(jnp.int32, (1, CBLK), 1)
    z = jnp.where(col < V, z, -jnp.inf)
    z_ref[...] = z[0]
    m_old = m_ref[0]
    m_new = jnp.maximum(m_old, jnp.max(z))
    bsum = jnp.sum(jnp.exp(z - m_new))
    s_ref[0] = s_ref[0] * jnp.exp(m_old - m_new) + bsum
    m_ref[0] = m_new

    @pl.when(g == NBLK - 1)
    def _():
        lse_ref[0] = m_ref[0] + jnp.log(s_ref[0])


def _norm_body(z_ref, lse_ref, o_ref):
    o_ref[...] = z_ref[...] - lse_ref[0]


def kernel(context, emb, W, b):
    ctx = context.astype(jnp.int32)
    embt = emb.T
    wt = W.T

    def _tile_spec(j):
        return pl.BlockSpec((D, 128), lambda i, c, j=j: (0, c[KG * i + j] // 128))

    e_wide = pl.pallas_call(
        _gather_body,
        grid_spec=pltpu.PrefetchScalarGridSpec(
            num_scalar_prefetch=1,
            grid=(NG,),
            in_specs=[_tile_spec(j) for j in range(KG)],
            out_specs=pl.BlockSpec((D, 128), lambda i, c: (0, 0)),
        ),
        out_shape=jax.ShapeDtypeStruct((D, 128), jnp.float32),
    )(ctx, *([embt] * KG))
    e2 = e_wide[:, :1]

    z, lse = pl.pallas_call(
        _matvec_body,
        grid=(NBLK,),
        in_specs=[
            pl.BlockSpec((D, 1), lambda g: (0, 0)),
            pl.BlockSpec((D, CBLK), lambda g: (0, g)),
            pl.BlockSpec((CBLK,), lambda g: (g,)),
        ],
        out_specs=[
            pl.BlockSpec((CBLK,), lambda g: (g,)),
            pl.BlockSpec(block_shape=(1,), index_map=lambda g: (0,),
                         memory_space=pltpu.SMEM),
        ],
        out_shape=[
            jax.ShapeDtypeStruct((V,), jnp.float32),
            jax.ShapeDtypeStruct((1,), jnp.float32),
        ],
        scratch_shapes=[
            pltpu.SMEM((1,), jnp.float32),
            pltpu.SMEM((1,), jnp.float32),
        ],
    )(e2, wt, b)

    out = pl.pallas_call(
        _norm_body,
        grid=(NBLK,),
        in_specs=[
            pl.BlockSpec((CBLK,), lambda g: (g,)),
            pl.BlockSpec(block_shape=(1,), index_map=lambda g: (0,),
                         memory_space=pltpu.SMEM),
        ],
        out_specs=pl.BlockSpec((CBLK,), lambda g: (g,)),
        out_shape=jax.ShapeDtypeStruct((V,), jnp.float32),
    )(z, lse)
    return out
